# baseline (speedup 1.0000x reference)
import functools

import jax
import jax.numpy as jnp
from jax import lax
from jax.experimental import pallas as pl
from jax.experimental.pallas import tpu as pltpu
from jax.experimental.pallas import tpu_sc as plsc

F = 26
V = 100000
D = 32
B = 16384
OUT = 128

NC, NS, LANES = 2, 16, 16
NW = NC * NS
BF = B * F
PER_W = BF // NW
CH = 128
N_CH = PER_W // CH


NB = V // 128
VREM = V - NB * 128
R2 = F * V // 4


def _transpose_sc(tables_t):
    mesh = plsc.VectorSubcoreMesh(core_axis_name="c", subcore_axis_name="s")

    @functools.partial(
        pl.kernel,
        mesh=mesh,
        out_type=jax.ShapeDtypeStruct((R2, 128), jnp.float32),
        compiler_params=pltpu.CompilerParams(
            use_tc_tiling_on_sc=True, needs_layout_passes=False),
        scratch_types=[
            pltpu.VMEM((D, 128), jnp.float32),
            pltpu.VMEM((D, 128), jnp.float32),
            pltpu.VMEM((D, VREM), jnp.float32),
            pltpu.VMEM((VREM // 4, 128), jnp.float32),
        ],
    )
    def k(tbl_hbm, out_hbm, ib, ob, ibp, obp):
        wid = lax.axis_index("s") * NC + lax.axis_index("c")
        iota = lax.iota(jnp.int32, LANES)

        def transpose_block(src, dst, nq):
            for q in range(nq):
                for s in range(8):
                    row = iota + 16 * (s & 1)
                    col = jnp.full((LANES,), 4 * q + s // 2, jnp.int32)
                    dst[q, pl.ds(16 * s, LANES)] = plsc.load_gather(
                        src, [row, col])

        def fbody(f, carry):
            def jbody(u, carry2):
                j = wid + u * NW

                @pl.when(j < NB)
                def _():
                    pltpu.sync_copy(tbl_hbm.at[f, :, pl.ds(j * 128, 128)], ib)
                    transpose_block(ib, ob, 32)
                    pltpu.sync_copy(
                        ob, out_hbm.at[pl.ds(f * (V // 4) + j * 32, 32)])

                return carry2

            lax.fori_loop(0, (NB + NW - 1) // NW, jbody, 0)

            @pl.when(f == wid)
            def _():
                pltpu.sync_copy(tbl_hbm.at[f, :, pl.ds(NB * 128, VREM)], ibp)
                transpose_block(ibp, obp, VREM // 4)
                pltpu.sync_copy(
                    obp, out_hbm.at[pl.ds(f * (V // 4) + NB * 32, VREM // 4)])

            return carry

        lax.fori_loop(0, F, fbody, 0)

    return k(tables_t)


def _gather_sc(x_flat, offs, table_flat):
    mesh = plsc.VectorSubcoreMesh(core_axis_name="c", subcore_axis_name="s")

    @functools.partial(
        pl.kernel,
        mesh=mesh,
        out_type=jax.ShapeDtypeStruct((BF, D), jnp.float32),
        compiler_params=pltpu.CompilerParams(use_tc_tiling_on_sc=False),
        scratch_types=[
            pltpu.VMEM((PER_W,), jnp.int32),
            pltpu.VMEM((PER_W,), jnp.int32),
            pltpu.VMEM((CH, D), jnp.float32),
            pltpu.SemaphoreType.DMA,
        ],
    )
    def k(x_hbm, offs_hbm, tbl_hbm, out_hbm, idx_v, offs_v, rows_v, sem):
        wid = lax.axis_index("s") * NC + lax.axis_index("c")
        base = wid * PER_W
        pltpu.sync_copy(x_hbm.at[pl.ds(base, PER_W)], idx_v)
        pltpu.sync_copy(offs_hbm, offs_v)

        def add_body(i, carry):
            s = pl.ds(i * LANES, LANES)
            idx_v[s] = idx_v[s] + offs_v[s]
            return carry

        lax.fori_loop(0, PER_W // LANES, add_body, 0)

        def gather_body(j, carry):
            pltpu.async_copy(
                tbl_hbm.at[idx_v.at[pl.ds(j * CH, CH)]], rows_v, sem
            ).wait()
            pltpu.sync_copy(rows_v, out_hbm.at[pl.ds(base + j * CH, CH)])
            return carry

        lax.fori_loop(0, N_CH, gather_body, 0)

    return k(x_flat, offs, table_flat)


def _matmul_tc(a, w):
    BM = 1024

    def body(a_ref, w_ref, o_ref):
        o_ref[...] = jnp.dot(a_ref[...], w_ref[...],
                             preferred_element_type=jnp.float32)

    return pl.pallas_call(
        body,
        grid=(B // BM,),
        in_specs=[
            pl.BlockSpec((BM, F * D), lambda i: (i, 0)),
            pl.BlockSpec((F * D, OUT), lambda i: (0, 0)),
        ],
        out_specs=pl.BlockSpec((BM, OUT), lambda i: (i, 0)),
        out_shape=jax.ShapeDtypeStruct((B, OUT), jnp.float32),
    )(a, w)


def kernel(x, tables, W):
    tables_t = jnp.transpose(tables, (0, 2, 1))
    table_flat = _transpose_sc(tables_t).reshape(F * V, D)
    x_flat = x.reshape(BF)
    offs = jnp.tile(jnp.arange(F, dtype=jnp.int32) * V, PER_W // F)
    gathered = _gather_sc(x_flat, offs, table_flat)
    return _matmul_tc(gathered.reshape(B, F * D), W)


# speedup vs baseline: 1.3207x; 1.3207x over previous
import functools

import jax
import jax.numpy as jnp
from jax import lax
from jax.experimental import pallas as pl
from jax.experimental.pallas import tpu as pltpu
from jax.experimental.pallas import tpu_sc as plsc

F = 26
V = 100000
D = 32
B = 16384
OUT = 128

NC, NS, LANES = 2, 16, 16
NW = NC * NS
BF = B * F
PER_W = BF // NW
CH = 128
N_CH = PER_W // CH


R2 = F * V // 4
BV = 512
NBIG = V // BV
NFULL = F * NBIG
VL = NBIG * BV
LW = V - VL
ROWS_PER_F = V // 4


def _transpose_sc(tables_t):
    mesh = plsc.VectorSubcoreMesh(core_axis_name="c", subcore_axis_name="s")

    @functools.partial(
        pl.kernel,
        mesh=mesh,
        out_type=jax.ShapeDtypeStruct((R2, 128), jnp.float32),
        compiler_params=pltpu.CompilerParams(
            use_tc_tiling_on_sc=True, needs_layout_passes=False),
        scratch_types=[
            pltpu.VMEM((2, 4, D, 128), jnp.float32),
            pltpu.VMEM((2, BV // 4, 128), jnp.float32),
            pltpu.VMEM((D, 128), jnp.float32),
            pltpu.VMEM((D, LW - 128), jnp.float32),
            pltpu.VMEM((LW // 4, 128), jnp.float32),
            pltpu.SemaphoreType.DMA,
            pltpu.SemaphoreType.DMA,
            pltpu.SemaphoreType.DMA,
            pltpu.SemaphoreType.DMA,
        ],
    )
    def k(tbl, out, ib, ob, ilp0, ilp1, olp, si0, si1, so0, so1):
        wid = lax.axis_index("s") * NC + lax.axis_index("c")
        iota = lax.iota(jnp.int32, LANES)
        iota16 = iota + 16
        sin = (si0, si1)
        sout = (so0, so1)
        nt = jnp.where(wid < NFULL % NW, NFULL // NW + 1, NFULL // NW)

        def adv2(f, j):
            j2 = j + 2 * NW
            w = (j2 >= NBIG).astype(jnp.int32)
            return f + w, j2 - w * NBIG

        def issue_in(b, f, j):
            for k4 in range(4):
                pltpu.async_copy(
                    tbl.at[f, :, pl.ds(j * BV + k4 * 128, 128)],
                    ib.at[b, k4], sin[b])

        def wait_in(b):
            for k4 in range(4):
                pltpu.make_async_copy(
                    tbl.at[0, :, pl.ds(0, 128)], ib.at[b, k4], sin[b]).wait()

        def issue_out(b, f, j):
            pltpu.async_copy(
                ob.at[b],
                out.at[pl.ds(f * ROWS_PER_F + j * (BV // 4), BV // 4)],
                sout[b])

        def wait_out(b):
            pltpu.make_async_copy(
                ob.at[b], out.at[pl.ds(0, BV // 4)], sout[b]).wait()

        def transpose_big(b):
            def qbody(q, cr):
                for u in range(4):
                    c = 4 * q + u
                    blkv = jnp.full((LANES,), c // 128, jnp.int32)
                    colv = jnp.full((LANES,), c % 128, jnp.int32)
                    ob[b, q, pl.ds(32 * u, LANES)] = plsc.load_gather(
                        ib.at[b], [blkv, iota, colv])
                    ob[b, q, pl.ds(32 * u + 16, LANES)] = plsc.load_gather(
                        ib.at[b], [blkv, iota16, colv])
                return cr

            lax.fori_loop(0, BV // 4, qbody, 0)

        issue_in(0, jnp.int32(0), wid)
        f1, j1 = jnp.int32(0), wid + NW
        issue_in(1, f1, j1)

        def ttbody(tt, carry):
            fA, jA, fB, jB = carry
            t0 = 2 * tt
            for b, fX, jX in ((0, fA, jA), (1, fB, jB)):
                t = t0 + b

                @pl.when(t < nt)
                def _():
                    wait_in(b)
                    transpose_big(b)

                    @pl.when(t >= 2)
                    def _():
                        wait_out(b)

                    issue_out(b, fX, jX)
                    fN, jN = adv2(fX, jX)

                    @pl.when(t + 2 < nt)
                    def _():
                        issue_in(b, fN, jN)

            fA2, jA2 = adv2(fA, jA)
            fB2, jB2 = adv2(fB, jB)
            return fA2, jA2, fB2, jB2

        lax.fori_loop(0, (NFULL // NW + 2) // 2, ttbody,
                      (jnp.int32(0), wid, f1, j1))
        wait_out(0)
        wait_out(1)

        @pl.when(wid < F)
        def _():
            f = wid
            pltpu.sync_copy(tbl.at[f, :, pl.ds(VL, 128)], ilp0)
            pltpu.sync_copy(tbl.at[f, :, pl.ds(VL + 128, LW - 128)], ilp1)
            for q in range(LW // 4):
                for u in range(4):
                    c = 4 * q + u
                    src = ilp0 if c < 128 else ilp1
                    cv = jnp.full((LANES,), c % 128, jnp.int32)
                    olp[q, pl.ds(32 * u, LANES)] = plsc.load_gather(
                        src, [iota, cv])
                    olp[q, pl.ds(32 * u + 16, LANES)] = plsc.load_gather(
                        src, [iota16, cv])
            pltpu.sync_copy(
                olp, out.at[pl.ds(f * ROWS_PER_F + VL // 4, LW // 4)])

    return k(tables_t)


def _gather_sc(x_flat, offs, table_flat):
    mesh = plsc.VectorSubcoreMesh(core_axis_name="c", subcore_axis_name="s")

    @functools.partial(
        pl.kernel,
        mesh=mesh,
        out_type=jax.ShapeDtypeStruct((BF, D), jnp.float32),
        compiler_params=pltpu.CompilerParams(use_tc_tiling_on_sc=False),
        scratch_types=[
            pltpu.VMEM((PER_W,), jnp.int32),
            pltpu.VMEM((PER_W,), jnp.int32),
            pltpu.VMEM((CH, D), jnp.float32),
            pltpu.SemaphoreType.DMA,
        ],
    )
    def k(x_hbm, offs_hbm, tbl_hbm, out_hbm, idx_v, offs_v, rows_v, sem):
        wid = lax.axis_index("s") * NC + lax.axis_index("c")
        base = wid * PER_W
        pltpu.sync_copy(x_hbm.at[pl.ds(base, PER_W)], idx_v)
        pltpu.sync_copy(offs_hbm, offs_v)

        def add_body(i, carry):
            s = pl.ds(i * LANES, LANES)
            idx_v[s] = idx_v[s] + offs_v[s]
            return carry

        lax.fori_loop(0, PER_W // LANES, add_body, 0)

        def gather_body(j, carry):
            pltpu.async_copy(
                tbl_hbm.at[idx_v.at[pl.ds(j * CH, CH)]], rows_v, sem
            ).wait()
            pltpu.sync_copy(rows_v, out_hbm.at[pl.ds(base + j * CH, CH)])
            return carry

        lax.fori_loop(0, N_CH, gather_body, 0)

    return k(x_flat, offs, table_flat)


def _matmul_tc(a, w):
    BM = 1024

    def body(a_ref, w_ref, o_ref):
        o_ref[...] = jnp.dot(a_ref[...], w_ref[...],
                             preferred_element_type=jnp.float32)

    return pl.pallas_call(
        body,
        grid=(B // BM,),
        in_specs=[
            pl.BlockSpec((BM, F * D), lambda i: (i, 0)),
            pl.BlockSpec((F * D, OUT), lambda i: (0, 0)),
        ],
        out_specs=pl.BlockSpec((BM, OUT), lambda i: (i, 0)),
        out_shape=jax.ShapeDtypeStruct((B, OUT), jnp.float32),
    )(a, w)


def kernel(x, tables, W):
    tables_t = jnp.transpose(tables, (0, 2, 1))
    table_flat = _transpose_sc(tables_t).reshape(F * V, D)
    x_flat = x.reshape(BF)
    offs = jnp.tile(jnp.arange(F, dtype=jnp.int32) * V, PER_W // F)
    gathered = _gather_sc(x_flat, offs, table_flat)
    return _matmul_tc(gathered.reshape(B, F * D), W)


# speedup vs baseline: 1.6020x; 1.2130x over previous
import functools

import jax
import jax.numpy as jnp
from jax import lax
from jax.experimental import pallas as pl
from jax.experimental.pallas import tpu as pltpu
from jax.experimental.pallas import tpu_sc as plsc

F = 26
V = 100000
D = 32
B = 16384
OUT = 128

NC, NS, LANES = 2, 16, 16
NW = NC * NS
BF = B * F
PER_W = BF // NW
CH = 128
N_CH = PER_W // CH


R2 = F * V // 4
BV = 512
NBIG = V // BV
NFULL = F * NBIG
VL = NBIG * BV
LW = V - VL
ROWS_PER_F = V // 4


def _transpose_sc(tables_t):
    mesh = plsc.VectorSubcoreMesh(core_axis_name="c", subcore_axis_name="s")

    @functools.partial(
        pl.kernel,
        mesh=mesh,
        out_type=jax.ShapeDtypeStruct((R2, 128), jnp.float32),
        compiler_params=pltpu.CompilerParams(
            use_tc_tiling_on_sc=True, needs_layout_passes=False),
        scratch_types=[
            pltpu.VMEM((2, 4, D, 128), jnp.float32),
            pltpu.VMEM((2, BV // 4, 128), jnp.float32),
            pltpu.VMEM((D, 128), jnp.float32),
            pltpu.VMEM((D, LW - 128), jnp.float32),
            pltpu.VMEM((LW // 4, 128), jnp.float32),
            pltpu.SemaphoreType.DMA,
            pltpu.SemaphoreType.DMA,
            pltpu.SemaphoreType.DMA,
            pltpu.SemaphoreType.DMA,
        ],
    )
    def k(tbl, out, ib, ob, ilp0, ilp1, olp, si0, si1, so0, so1):
        wid = lax.axis_index("s") * NC + lax.axis_index("c")
        iota = lax.iota(jnp.int32, LANES)
        iota16 = iota + 16
        sin = (si0, si1)
        sout = (so0, so1)
        nt = jnp.where(wid < NFULL % NW, NFULL // NW + 1, NFULL // NW)

        def adv2(f, j):
            j2 = j + 2 * NW
            w = (j2 >= NBIG).astype(jnp.int32)
            return f + w, j2 - w * NBIG

        def issue_in(b, f, j):
            for k4 in range(4):
                pltpu.async_copy(
                    tbl.at[f, :, pl.ds(j * BV + k4 * 128, 128)],
                    ib.at[b, k4], sin[b])

        def wait_in(b):
            for k4 in range(4):
                pltpu.make_async_copy(
                    tbl.at[0, :, pl.ds(0, 128)], ib.at[b, k4], sin[b]).wait()

        def issue_out(b, f, j):
            pltpu.async_copy(
                ob.at[b],
                out.at[pl.ds(f * ROWS_PER_F + j * (BV // 4), BV // 4)],
                sout[b])

        def wait_out(b):
            pltpu.make_async_copy(
                ob.at[b], out.at[pl.ds(0, BV // 4)], sout[b]).wait()

        colpat = (iota % 4) * 32
        rowbase = iota // 4

        def transpose_big(b):
            def dbody(d, cr):
                colv = colpat + d
                for k4 in range(4):
                    for m in range(8):
                        rowv = rowbase + (32 * k4 + 4 * m)
                        val = ib[b, k4, d, pl.ds(16 * m, LANES)]
                        plsc.store_scatter(ob.at[b], [rowv, colv], val)
                return cr

            lax.fori_loop(0, D, dbody, 0)

        issue_in(0, jnp.int32(0), wid)
        f1, j1 = jnp.int32(0), wid + NW
        issue_in(1, f1, j1)

        def ttbody(tt, carry):
            fA, jA, fB, jB = carry
            t0 = 2 * tt
            for b, fX, jX in ((0, fA, jA), (1, fB, jB)):
                t = t0 + b

                @pl.when(t < nt)
                def _():
                    wait_in(b)
                    transpose_big(b)

                    @pl.when(t >= 2)
                    def _():
                        wait_out(b)

                    issue_out(b, fX, jX)
                    fN, jN = adv2(fX, jX)

                    @pl.when(t + 2 < nt)
                    def _():
                        issue_in(b, fN, jN)

            fA2, jA2 = adv2(fA, jA)
            fB2, jB2 = adv2(fB, jB)
            return fA2, jA2, fB2, jB2

        lax.fori_loop(0, (NFULL // NW + 2) // 2, ttbody,
                      (jnp.int32(0), wid, f1, j1))
        wait_out(0)
        wait_out(1)

        @pl.when(wid < F)
        def _():
            f = wid
            pltpu.sync_copy(tbl.at[f, :, pl.ds(VL, 128)], ilp0)
            pltpu.sync_copy(tbl.at[f, :, pl.ds(VL + 128, LW - 128)], ilp1)
            for q in range(LW // 4):
                for u in range(4):
                    c = 4 * q + u
                    src = ilp0 if c < 128 else ilp1
                    cv = jnp.full((LANES,), c % 128, jnp.int32)
                    olp[q, pl.ds(32 * u, LANES)] = plsc.load_gather(
                        src, [iota, cv])
                    olp[q, pl.ds(32 * u + 16, LANES)] = plsc.load_gather(
                        src, [iota16, cv])
            pltpu.sync_copy(
                olp, out.at[pl.ds(f * ROWS_PER_F + VL // 4, LW // 4)])

    return k(tables_t)


def _gather_sc(x_flat, offs, table_flat):
    mesh = plsc.VectorSubcoreMesh(core_axis_name="c", subcore_axis_name="s")

    @functools.partial(
        pl.kernel,
        mesh=mesh,
        out_type=jax.ShapeDtypeStruct((BF, D), jnp.float32),
        compiler_params=pltpu.CompilerParams(use_tc_tiling_on_sc=False),
        scratch_types=[
            pltpu.VMEM((PER_W,), jnp.int32),
            pltpu.VMEM((PER_W,), jnp.int32),
            pltpu.VMEM((CH, D), jnp.float32),
            pltpu.SemaphoreType.DMA,
        ],
    )
    def k(x_hbm, offs_hbm, tbl_hbm, out_hbm, idx_v, offs_v, rows_v, sem):
        wid = lax.axis_index("s") * NC + lax.axis_index("c")
        base = wid * PER_W
        pltpu.sync_copy(x_hbm.at[pl.ds(base, PER_W)], idx_v)
        pltpu.sync_copy(offs_hbm, offs_v)

        def add_body(i, carry):
            s = pl.ds(i * LANES, LANES)
            idx_v[s] = idx_v[s] + offs_v[s]
            return carry

        lax.fori_loop(0, PER_W // LANES, add_body, 0)

        def gather_body(j, carry):
            pltpu.async_copy(
                tbl_hbm.at[idx_v.at[pl.ds(j * CH, CH)]], rows_v, sem
            ).wait()
            pltpu.sync_copy(rows_v, out_hbm.at[pl.ds(base + j * CH, CH)])
            return carry

        lax.fori_loop(0, N_CH, gather_body, 0)

    return k(x_flat, offs, table_flat)


def _matmul_tc(a, w):
    BM = 1024

    def body(a_ref, w_ref, o_ref):
        o_ref[...] = jnp.dot(a_ref[...], w_ref[...],
                             preferred_element_type=jnp.float32)

    return pl.pallas_call(
        body,
        grid=(B // BM,),
        in_specs=[
            pl.BlockSpec((BM, F * D), lambda i: (i, 0)),
            pl.BlockSpec((F * D, OUT), lambda i: (0, 0)),
        ],
        out_specs=pl.BlockSpec((BM, OUT), lambda i: (i, 0)),
        out_shape=jax.ShapeDtypeStruct((B, OUT), jnp.float32),
    )(a, w)


def kernel(x, tables, W):
    tables_t = jnp.transpose(tables, (0, 2, 1))
    table_flat = _transpose_sc(tables_t).reshape(F * V, D)
    x_flat = x.reshape(BF)
    offs = jnp.tile(jnp.arange(F, dtype=jnp.int32) * V, PER_W // F)
    gathered = _gather_sc(x_flat, offs, table_flat)
    return _matmul_tc(gathered.reshape(B, F * D), W)
